# baseline (speedup 1.0000x reference)
import jax
import jax.numpy as jnp
from jax.experimental import pallas as pl

_SIDE = 14
_NUM = 3
_CLASSES = 20
_CELLS = _SIDE * _SIDE
_CH = _NUM * 5 + _CLASSES
_FEAT = _CELLS * _CH
_BOXES = _CELLS * _NUM
_BT = 128


def _decode_body(x_ref, cls_ref, loc_ref, conf_ref):
    xt = x_ref[...].T
    e = jnp.exp(xt)
    sg = e / (1.0 + e)
    e3 = e.reshape(_CELLS, _CH, _BT)
    s3 = sg.reshape(_CELLS, _CH, _BT)

    ecls = e3[:, _NUM * 5:, :]
    rec = 1.0 / jnp.sum(ecls, axis=1, keepdims=True)

    r_i = jax.lax.broadcasted_iota(jnp.int32, (_BOXES, _CELLS), 0)
    c_i = jax.lax.broadcasted_iota(jnp.int32, (_BOXES, _CELLS), 1)
    r3 = (r_i // _NUM == c_i).astype(jnp.float32)

    for j in range(_CLASSES):
        pj = ecls[:, j, :] * rec[:, 0, :]
        cls_ref[j, :, :] = jnp.dot(r3, pj, preferred_element_type=jnp.float32)

    c_idx = jax.lax.broadcasted_iota(jnp.int32, (_CELLS, 1, 1), 0)
    gx = (c_idx % _SIDE).astype(jnp.float32)
    gy = (c_idx // _SIDE).astype(jnp.float32)
    grid = jnp.concatenate([gx, gy], axis=1)

    las = []
    cfs = []
    for a in range(_NUM):
        base = a * 5
        pxy = (s3[:, base:base + 2, :] + grid) * (1.0 / _SIDE)
        wh = s3[:, base + 2:base + 4, :]
        las.append(jnp.concatenate([pxy, wh], axis=1))
        cfs.append(s3[:, base + 4, :])
    loc_ref[...] = jnp.stack(las, axis=1).reshape(_BOXES * 4, _BT)
    conf_ref[...] = jnp.stack(cfs, axis=1).reshape(_BOXES, _BT)


def kernel(b_x):
    bsz = b_x.shape[0]
    cls_t, loc_t, conf_t = pl.pallas_call(
        _decode_body,
        grid=(bsz // _BT,),
        in_specs=[pl.BlockSpec((_BT, _FEAT), lambda i: (i, 0))],
        out_specs=(
            pl.BlockSpec((_CLASSES, _BOXES, _BT), lambda i: (0, 0, i)),
            pl.BlockSpec((_BOXES * 4, _BT), lambda i: (0, i)),
            pl.BlockSpec((_BOXES, _BT), lambda i: (0, i)),
        ),
        out_shape=(
            jax.ShapeDtypeStruct((_CLASSES, _BOXES, bsz), b_x.dtype),
            jax.ShapeDtypeStruct((_BOXES * 4, bsz), b_x.dtype),
            jax.ShapeDtypeStruct((_BOXES, bsz), b_x.dtype),
        ),
    )(b_x)
    return (
        loc_t.reshape(_BOXES, 4, bsz).transpose(2, 0, 1),
        cls_t.transpose(2, 1, 0),
        conf_t.T,
    )


# speedup vs baseline: 1.7056x; 1.7056x over previous
import jax
import jax.numpy as jnp
from jax.experimental import pallas as pl

_SIDE = 14
_NUM = 3
_CLASSES = 20
_CELLS = _SIDE * _SIDE
_CH = _NUM * 5 + _CLASSES
_FEAT = _CELLS * _CH
_BOXES = _CELLS * _NUM
_BT = 128


def _decode_body(x_ref, cls_ref, loc_ref, conf_ref):
    xt = x_ref[...]
    e = jnp.exp(xt)
    sg = e / (1.0 + e)
    e3 = e.reshape(_CELLS, _CH, _BT)
    s3 = sg.reshape(_CELLS, _CH, _BT)

    ecls = e3[:, _NUM * 5:, :]
    rec = 1.0 / jnp.sum(ecls, axis=1, keepdims=True)

    r_i = jax.lax.broadcasted_iota(jnp.int32, (_BOXES, _CELLS), 0)
    c_i = jax.lax.broadcasted_iota(jnp.int32, (_BOXES, _CELLS), 1)
    r3 = (r_i // _NUM == c_i).astype(jnp.float32)

    for j in range(_CLASSES):
        pj = ecls[:, j, :] * rec[:, 0, :]
        cls_ref[j, :, :] = jnp.dot(r3, pj, preferred_element_type=jnp.float32)

    c_idx = jax.lax.broadcasted_iota(jnp.int32, (_CELLS, 1, 1), 0)
    gx = (c_idx % _SIDE).astype(jnp.float32)
    gy = (c_idx // _SIDE).astype(jnp.float32)
    grid = jnp.concatenate([gx, gy], axis=1)

    las = []
    cfs = []
    for a in range(_NUM):
        base = a * 5
        pxy = (s3[:, base:base + 2, :] + grid) * (1.0 / _SIDE)
        wh = s3[:, base + 2:base + 4, :]
        las.append(jnp.concatenate([pxy, wh], axis=1))
        cfs.append(s3[:, base + 4, :])
    loc_ref[...] = jnp.stack(las, axis=1).reshape(_BOXES, 4, _BT)
    conf_ref[...] = jnp.stack(cfs, axis=1).reshape(_BOXES, _BT)


def kernel(b_x):
    bsz = b_x.shape[0]
    xt_in = b_x.T
    cls_t, loc_t, conf_t = pl.pallas_call(
        _decode_body,
        grid=(bsz // _BT,),
        in_specs=[pl.BlockSpec((_FEAT, _BT), lambda i: (0, i))],
        out_specs=(
            pl.BlockSpec((_CLASSES, _BOXES, _BT), lambda i: (0, 0, i)),
            pl.BlockSpec((_BOXES, 4, _BT), lambda i: (0, 0, i)),
            pl.BlockSpec((_BOXES, _BT), lambda i: (0, i)),
        ),
        out_shape=(
            jax.ShapeDtypeStruct((_CLASSES, _BOXES, bsz), b_x.dtype),
            jax.ShapeDtypeStruct((_BOXES, 4, bsz), b_x.dtype),
            jax.ShapeDtypeStruct((_BOXES, bsz), b_x.dtype),
        ),
    )(xt_in)
    return (
        loc_t.transpose(2, 0, 1),
        cls_t.transpose(2, 1, 0),
        conf_t.T,
    )
